# baseline (speedup 1.0000x reference)
import jax
import jax.numpy as jnp
from jax import lax
from jax.experimental import pallas as pl
from jax.experimental.pallas import tpu as pltpu
from jax.experimental.pallas import tpu_sc as plsc

NC = 1
NS = 16
NW = NC * NS
L = 16
RB = 16
D = 512
CHUNKS = D // L

TC_ROWS = 8704
BR = 512


def _sc_body(remap_hbm, emb_hbm, nl_hbm, out_hbm,
             nl_ref, acc_ref, mat_ref,
             br0, be0, br1, be1, br2, be2, br3, be3,
             sem0, sem1, sem2, sem3):
    cid = lax.axis_index("c")
    sid = lax.axis_index("s")
    wid = cid * NS + sid

    pltpu.sync_copy(nl_hbm, nl_ref)
    nl = nl_ref[...]
    wseg_vec = 1.0 / jnp.maximum(nl, 1).astype(jnp.float32)
    cum = []
    run = jnp.int32(0)
    wseg = []
    for j in range(16):
        run = run + nl[j]
        cum.append(run)
        wseg.append(wseg_vec[j])
    total = cum[15]
    rem = jnp.maximum(total - TC_ROWS, 0)
    nb = (rem + RB - 1) >> 4
    nmy = (jnp.maximum(nb - wid, 0) + (NW - 1)) // NW

    def copies(i, br, be, sem):
        row0 = TC_ROWS + (wid + i * NW) * RB
        cr = pltpu.make_async_copy(remap_hbm.at[pl.ds(row0, RB)], br, sem)
        ce = pltpu.make_async_copy(emb_hbm.at[pl.ds(row0, RB)], be, sem)
        return cr, ce

    def issue(i, br, be, sem):
        cr, ce = copies(i, br, be, sem)
        cr.start()
        ce.start()

    def drain(i, br, be, sem):
        cr, ce = copies(i, br, be, sem)
        cr.wait()
        ce.wait()

    lane = lax.iota(jnp.int32, L)

    def compute_block(br, be, i):
        row0 = TC_ROWS + (wid + i * NW) * RB

        def row_body(r, carry):
            a16 = jnp.zeros((L,), jnp.float32)
            for c in range(CHUNKS):
                a = br[r, pl.ds(c * L, L)]
                b = be[r, pl.ds(c * L, L)]
                d = a - b
                a16 = a16 + d * d
            mat_ref[r, :] = a16
            return carry
        lax.fori_loop(0, RB, row_body, 0, unroll=2)

        ssq = jnp.zeros((L,), jnp.float32)
        for c in range(L):
            col = jnp.zeros((L,), jnp.int32) + c
            ssq = ssq + plsc.load_gather(mat_ref, [lane, col])

        s = jnp.maximum(ssq, 1e-30)
        ii = plsc.bitcast(s, jnp.int32)
        y = plsc.bitcast(jnp.int32(0x5F3759DF) - (ii >> 1), jnp.float32)
        for _ in range(3):
            y = y * (1.5 - 0.5 * s * y * y)
        norm = s * y

        p = row0 + lane
        w = jnp.zeros((L,), jnp.float32) + wseg[0]
        for j in range(1, 16):
            w = w + jnp.where(p >= cum[j - 1], wseg[j] - wseg[j - 1], 0.0)
        w = jnp.where(p >= total, 0.0, w)
        return w * norm

    slots = ((br0, be0, sem0), (br1, be1, sem1),
             (br2, be2, sem2), (br3, be3, sem3))

    for s in range(3):
        @pl.when(s < nmy)
        def _(s=s):
            issue(s, *slots[s])

    def sub_block(i, u, acc):
        br, be, sem = slots[u]

        def go(a):
            drain(i, br, be, sem)

            @pl.when(i + 3 < nmy)
            def _():
                issue(i + 3, *slots[(u + 3) % 4])

            return a + compute_block(br, be, i)

        return lax.cond(i < nmy, go, lambda a: a, acc)

    def quad_body(q, acc):
        i0 = 4 * q
        for u in range(4):
            acc = sub_block(i0 + u, u, acc)
        return acc

    nquads = (nmy + 3) >> 2
    acc = lax.fori_loop(0, nquads, quad_body, jnp.zeros((L,), jnp.float32))

    acc_ref[...] = acc
    pltpu.sync_copy(acc_ref, out_hbm.at[wid])


def _tc_body(nl_ref, a_ref, b_ref, o_ref):
    i = pl.program_id(0)
    d = a_ref[...] - b_ref[...]
    s = jnp.sum(d * d, axis=1).reshape(BR // 128, 128)
    norm = jnp.sqrt(s)

    p = (i * BR
         + lax.broadcasted_iota(jnp.int32, (BR // 128, 128), 0) * 128
         + lax.broadcasted_iota(jnp.int32, (BR // 128, 128), 1))

    w = jnp.full((BR // 128, 128), 1.0 / jnp.maximum(nl_ref[0], 1).astype(jnp.float32))
    off = nl_ref[0]
    for j in range(1, 16):
        wj = 1.0 / jnp.maximum(nl_ref[j], 1).astype(jnp.float32)
        w = jnp.where(p >= off, wj, w)
        off = off + nl_ref[j]
    w = jnp.where(p >= off, 0.0, w)

    @pl.when(i == 0)
    def _():
        o_ref[...] = jnp.zeros_like(o_ref)

    o_ref[...] += jnp.sum(w * norm).reshape(1, 1)


@jax.jit
def _combined(clip_remap, clip_emb, num_list):
    mesh = plsc.VectorSubcoreMesh(core_axis_name="c", subcore_axis_name="s",
                                  num_cores=NC, num_subcores=NS)
    sc = pl.kernel(
        _sc_body,
        out_type=jax.ShapeDtypeStruct((NW, L), jnp.float32),
        mesh=mesh,
        compiler_params=pltpu.CompilerParams(needs_layout_passes=False),
        scratch_types=[
            pltpu.VMEM((16,), jnp.int32),
            pltpu.VMEM((L,), jnp.float32),
            pltpu.VMEM((RB, L), jnp.float32),
            pltpu.VMEM((RB, D), jnp.float32),
            pltpu.VMEM((RB, D), jnp.float32),
            pltpu.VMEM((RB, D), jnp.float32),
            pltpu.VMEM((RB, D), jnp.float32),
            pltpu.VMEM((RB, D), jnp.float32),
            pltpu.VMEM((RB, D), jnp.float32),
            pltpu.VMEM((RB, D), jnp.float32),
            pltpu.VMEM((RB, D), jnp.float32),
            pltpu.SemaphoreType.DMA,
            pltpu.SemaphoreType.DMA,
            pltpu.SemaphoreType.DMA,
            pltpu.SemaphoreType.DMA,
        ],
    )(clip_remap, clip_emb, num_list)

    tc = pl.pallas_call(
        _tc_body,
        grid=(TC_ROWS // BR,),
        in_specs=[
            pl.BlockSpec(memory_space=pltpu.SMEM),
            pl.BlockSpec((BR, D), lambda i: (i, 0)),
            pl.BlockSpec((BR, D), lambda i: (i, 0)),
        ],
        out_specs=pl.BlockSpec((1, 1), lambda i: (0, 0)),
        out_shape=jax.ShapeDtypeStruct((1, 1), jnp.float32),
        compiler_params=pltpu.CompilerParams(
            dimension_semantics=("arbitrary",)),
    )(num_list, clip_remap, clip_emb)

    return tc[0, 0] + jnp.sum(sc)


def kernel(clip_remap, clip_emb, num_list):
    return _combined(clip_remap, clip_emb, num_list)


# speedup vs baseline: 1.1194x; 1.1194x over previous
import jax
import jax.numpy as jnp
from jax import lax
from jax.experimental import pallas as pl
from jax.experimental.pallas import tpu as pltpu
from jax.experimental.pallas import tpu_sc as plsc

NC = 2
NS = 16
NW = NC * NS
L = 16
RB = 16
D = 512
CHUNKS = D // L

TC_ROWS = 10240
BR = 512


def _sc_body(remap_hbm, emb_hbm, nl_hbm, out_hbm,
             nl_ref, acc_ref, mat_ref,
             br0, be0, br1, be1, br2, be2, br3, be3,
             sem0, sem1, sem2, sem3):
    cid = lax.axis_index("c")
    sid = lax.axis_index("s")
    wid = cid * NS + sid

    pltpu.sync_copy(nl_hbm, nl_ref)
    nl = nl_ref[...]
    wseg_vec = 1.0 / jnp.maximum(nl, 1).astype(jnp.float32)
    cum = []
    run = jnp.int32(0)
    wseg = []
    for j in range(16):
        run = run + nl[j]
        cum.append(run)
        wseg.append(wseg_vec[j])
    total = cum[15]
    rem = jnp.maximum(total - TC_ROWS, 0)
    nb = (rem + RB - 1) >> 4
    nmy = (jnp.maximum(nb - wid, 0) + (NW - 1)) // NW

    def copies(i, br, be, sem):
        row0 = TC_ROWS + (wid + i * NW) * RB
        cr = pltpu.make_async_copy(remap_hbm.at[pl.ds(row0, RB)], br, sem)
        ce = pltpu.make_async_copy(emb_hbm.at[pl.ds(row0, RB)], be, sem)
        return cr, ce

    def issue(i, br, be, sem):
        cr, ce = copies(i, br, be, sem)
        cr.start()
        ce.start()

    def drain(i, br, be, sem):
        cr, ce = copies(i, br, be, sem)
        cr.wait()
        ce.wait()

    lane = lax.iota(jnp.int32, L)

    def compute_block(br, be, i):
        row0 = TC_ROWS + (wid + i * NW) * RB

        def row_body(r, carry):
            a16 = jnp.zeros((L,), jnp.float32)
            for c in range(CHUNKS):
                a = br[r, pl.ds(c * L, L)]
                b = be[r, pl.ds(c * L, L)]
                d = a - b
                a16 = a16 + d * d
            mat_ref[r, :] = a16
            return carry
        lax.fori_loop(0, RB, row_body, 0, unroll=2)

        ssq = jnp.zeros((L,), jnp.float32)
        for c in range(L):
            col = jnp.zeros((L,), jnp.int32) + c
            ssq = ssq + plsc.load_gather(mat_ref, [lane, col])

        s = jnp.maximum(ssq, 1e-30)
        ii = plsc.bitcast(s, jnp.int32)
        y = plsc.bitcast(jnp.int32(0x5F3759DF) - (ii >> 1), jnp.float32)
        for _ in range(3):
            y = y * (1.5 - 0.5 * s * y * y)
        norm = s * y

        p = row0 + lane
        w = jnp.zeros((L,), jnp.float32) + wseg[0]
        for j in range(1, 16):
            w = w + jnp.where(p >= cum[j - 1], wseg[j] - wseg[j - 1], 0.0)
        w = jnp.where(p >= total, 0.0, w)
        return w * norm

    slots = ((br0, be0, sem0), (br1, be1, sem1),
             (br2, be2, sem2), (br3, be3, sem3))

    for s in range(3):
        @pl.when(s < nmy)
        def _(s=s):
            issue(s, *slots[s])

    def sub_block(i, u, acc):
        br, be, sem = slots[u]

        def go(a):
            drain(i, br, be, sem)

            @pl.when(i + 3 < nmy)
            def _():
                issue(i + 3, *slots[(u + 3) % 4])

            return a + compute_block(br, be, i)

        return lax.cond(i < nmy, go, lambda a: a, acc)

    def quad_body(q, acc):
        i0 = 4 * q
        for u in range(4):
            acc = sub_block(i0 + u, u, acc)
        return acc

    nquads = (nmy + 3) >> 2
    acc = lax.fori_loop(0, nquads, quad_body, jnp.zeros((L,), jnp.float32))

    acc_ref[...] = acc
    pltpu.sync_copy(acc_ref, out_hbm.at[wid])


def _tc_body(nl_ref, a_ref, b_ref, o_ref):
    i = pl.program_id(0)
    d = a_ref[...] - b_ref[...]
    s = jnp.sum(d * d, axis=1).reshape(BR // 128, 128)
    norm = jnp.sqrt(s)

    p = (i * BR
         + lax.broadcasted_iota(jnp.int32, (BR // 128, 128), 0) * 128
         + lax.broadcasted_iota(jnp.int32, (BR // 128, 128), 1))

    w = jnp.full((BR // 128, 128), 1.0 / jnp.maximum(nl_ref[0], 1).astype(jnp.float32))
    off = nl_ref[0]
    for j in range(1, 16):
        wj = 1.0 / jnp.maximum(nl_ref[j], 1).astype(jnp.float32)
        w = jnp.where(p >= off, wj, w)
        off = off + nl_ref[j]
    w = jnp.where(p >= off, 0.0, w)

    @pl.when(i == 0)
    def _():
        o_ref[...] = jnp.zeros_like(o_ref)

    o_ref[...] += jnp.sum(w * norm).reshape(1, 1)


@jax.jit
def _combined(clip_remap, clip_emb, num_list):
    mesh = plsc.VectorSubcoreMesh(core_axis_name="c", subcore_axis_name="s",
                                  num_cores=NC, num_subcores=NS)
    sc = pl.kernel(
        _sc_body,
        out_type=jax.ShapeDtypeStruct((NW, L), jnp.float32),
        mesh=mesh,
        compiler_params=pltpu.CompilerParams(needs_layout_passes=False),
        scratch_types=[
            pltpu.VMEM((16,), jnp.int32),
            pltpu.VMEM((L,), jnp.float32),
            pltpu.VMEM((RB, L), jnp.float32),
            pltpu.VMEM((RB, D), jnp.float32),
            pltpu.VMEM((RB, D), jnp.float32),
            pltpu.VMEM((RB, D), jnp.float32),
            pltpu.VMEM((RB, D), jnp.float32),
            pltpu.VMEM((RB, D), jnp.float32),
            pltpu.VMEM((RB, D), jnp.float32),
            pltpu.VMEM((RB, D), jnp.float32),
            pltpu.VMEM((RB, D), jnp.float32),
            pltpu.SemaphoreType.DMA,
            pltpu.SemaphoreType.DMA,
            pltpu.SemaphoreType.DMA,
            pltpu.SemaphoreType.DMA,
        ],
    )(clip_remap, clip_emb, num_list)

    tc = pl.pallas_call(
        _tc_body,
        grid=(TC_ROWS // BR,),
        in_specs=[
            pl.BlockSpec(memory_space=pltpu.SMEM),
            pl.BlockSpec((BR, D), lambda i: (i, 0)),
            pl.BlockSpec((BR, D), lambda i: (i, 0)),
        ],
        out_specs=pl.BlockSpec((1, 1), lambda i: (0, 0)),
        out_shape=jax.ShapeDtypeStruct((1, 1), jnp.float32),
        compiler_params=pltpu.CompilerParams(
            dimension_semantics=("arbitrary",)),
    )(num_list, clip_remap, clip_emb)

    return tc[0, 0] + jnp.sum(sc)


def kernel(clip_remap, clip_emb, num_list):
    return _combined(clip_remap, clip_emb, num_list)


# speedup vs baseline: 1.1334x; 1.0125x over previous
import jax
import jax.numpy as jnp
from jax import lax
from jax.experimental import pallas as pl
from jax.experimental.pallas import tpu as pltpu
from jax.experimental.pallas import tpu_sc as plsc

NC = 2
NS = 16
NW = NC * NS
L = 16
RB = 16
D = 512
CHUNKS = D // L

TC_ROWS = 9728
BR = 512


def _sc_body(remap_hbm, emb_hbm, nl_hbm, out_hbm,
             nl_ref, acc_ref, mat_ref,
             br0, be0, br1, be1, br2, be2, br3, be3,
             sem0, sem1, sem2, sem3):
    cid = lax.axis_index("c")
    sid = lax.axis_index("s")
    wid = cid * NS + sid

    pltpu.sync_copy(nl_hbm, nl_ref)
    nl = nl_ref[...]
    wseg_vec = 1.0 / jnp.maximum(nl, 1).astype(jnp.float32)
    cum = []
    run = jnp.int32(0)
    wseg = []
    for j in range(16):
        run = run + nl[j]
        cum.append(run)
        wseg.append(wseg_vec[j])
    total = cum[15]
    rem = jnp.maximum(total - TC_ROWS, 0)
    nb = (rem + RB - 1) >> 4
    nmy = (jnp.maximum(nb - wid, 0) + (NW - 1)) // NW

    def copies(i, br, be, sem):
        row0 = TC_ROWS + (wid + i * NW) * RB
        cr = pltpu.make_async_copy(remap_hbm.at[pl.ds(row0, RB)], br, sem)
        ce = pltpu.make_async_copy(emb_hbm.at[pl.ds(row0, RB)], be, sem)
        return cr, ce

    def issue(i, br, be, sem):
        cr, ce = copies(i, br, be, sem)
        cr.start()
        ce.start()

    def drain(i, br, be, sem):
        cr, ce = copies(i, br, be, sem)
        cr.wait()
        ce.wait()

    lane = lax.iota(jnp.int32, L)

    def compute_block(br, be, i):
        row0 = TC_ROWS + (wid + i * NW) * RB

        def row_body(r, carry):
            a16 = jnp.zeros((L,), jnp.float32)
            for c in range(CHUNKS):
                a = br[r, pl.ds(c * L, L)]
                b = be[r, pl.ds(c * L, L)]
                d = a - b
                a16 = a16 + d * d
            mat_ref[r, :] = a16
            return carry
        lax.fori_loop(0, RB, row_body, 0, unroll=2)

        ssq = jnp.zeros((L,), jnp.float32)
        for c in range(L):
            col = jnp.zeros((L,), jnp.int32) + c
            ssq = ssq + plsc.load_gather(mat_ref, [lane, col])

        s = jnp.maximum(ssq, 1e-30)
        ii = plsc.bitcast(s, jnp.int32)
        y = plsc.bitcast(jnp.int32(0x5F3759DF) - (ii >> 1), jnp.float32)
        for _ in range(3):
            y = y * (1.5 - 0.5 * s * y * y)
        norm = s * y

        p = row0 + lane
        w = jnp.zeros((L,), jnp.float32) + wseg[0]
        for j in range(1, 16):
            w = w + jnp.where(p >= cum[j - 1], wseg[j] - wseg[j - 1], 0.0)
        w = jnp.where(p >= total, 0.0, w)
        return w * norm

    slots = ((br0, be0, sem0), (br1, be1, sem1),
             (br2, be2, sem2), (br3, be3, sem3))

    for s in range(3):
        @pl.when(s < nmy)
        def _(s=s):
            issue(s, *slots[s])

    def sub_block(i, u, acc):
        br, be, sem = slots[u]

        def go(a):
            drain(i, br, be, sem)

            @pl.when(i + 3 < nmy)
            def _():
                issue(i + 3, *slots[(u + 3) % 4])

            return a + compute_block(br, be, i)

        return lax.cond(i < nmy, go, lambda a: a, acc)

    def quad_body(q, acc):
        i0 = 4 * q
        for u in range(4):
            acc = sub_block(i0 + u, u, acc)
        return acc

    nquads = (nmy + 3) >> 2
    acc = lax.fori_loop(0, nquads, quad_body, jnp.zeros((L,), jnp.float32))

    acc_ref[...] = acc
    pltpu.sync_copy(acc_ref, out_hbm.at[wid])


def _tc_body(nl_ref, a_ref, b_ref, o_ref):
    i = pl.program_id(0)
    d = a_ref[...] - b_ref[...]
    s = jnp.sum(d * d, axis=1).reshape(BR // 128, 128)
    norm = jnp.sqrt(s)

    p = (i * BR
         + lax.broadcasted_iota(jnp.int32, (BR // 128, 128), 0) * 128
         + lax.broadcasted_iota(jnp.int32, (BR // 128, 128), 1))

    w = jnp.full((BR // 128, 128), 1.0 / jnp.maximum(nl_ref[0], 1).astype(jnp.float32))
    off = nl_ref[0]
    for j in range(1, 16):
        wj = 1.0 / jnp.maximum(nl_ref[j], 1).astype(jnp.float32)
        w = jnp.where(p >= off, wj, w)
        off = off + nl_ref[j]
    w = jnp.where(p >= off, 0.0, w)

    @pl.when(i == 0)
    def _():
        o_ref[...] = jnp.zeros_like(o_ref)

    o_ref[...] += jnp.sum(w * norm).reshape(1, 1)


@jax.jit
def _combined(clip_remap, clip_emb, num_list):
    mesh = plsc.VectorSubcoreMesh(core_axis_name="c", subcore_axis_name="s",
                                  num_cores=NC, num_subcores=NS)
    sc = pl.kernel(
        _sc_body,
        out_type=jax.ShapeDtypeStruct((NW, L), jnp.float32),
        mesh=mesh,
        compiler_params=pltpu.CompilerParams(needs_layout_passes=False),
        scratch_types=[
            pltpu.VMEM((16,), jnp.int32),
            pltpu.VMEM((L,), jnp.float32),
            pltpu.VMEM((RB, L), jnp.float32),
            pltpu.VMEM((RB, D), jnp.float32),
            pltpu.VMEM((RB, D), jnp.float32),
            pltpu.VMEM((RB, D), jnp.float32),
            pltpu.VMEM((RB, D), jnp.float32),
            pltpu.VMEM((RB, D), jnp.float32),
            pltpu.VMEM((RB, D), jnp.float32),
            pltpu.VMEM((RB, D), jnp.float32),
            pltpu.VMEM((RB, D), jnp.float32),
            pltpu.SemaphoreType.DMA,
            pltpu.SemaphoreType.DMA,
            pltpu.SemaphoreType.DMA,
            pltpu.SemaphoreType.DMA,
        ],
    )(clip_remap, clip_emb, num_list)

    tc = pl.pallas_call(
        _tc_body,
        grid=(TC_ROWS // BR,),
        in_specs=[
            pl.BlockSpec(memory_space=pltpu.SMEM),
            pl.BlockSpec((BR, D), lambda i: (i, 0)),
            pl.BlockSpec((BR, D), lambda i: (i, 0)),
        ],
        out_specs=pl.BlockSpec((1, 1), lambda i: (0, 0)),
        out_shape=jax.ShapeDtypeStruct((1, 1), jnp.float32),
        compiler_params=pltpu.CompilerParams(
            dimension_semantics=("arbitrary",)),
    )(num_list, clip_remap, clip_emb)

    return tc[0, 0] + jnp.sum(sc)


def kernel(clip_remap, clip_emb, num_list):
    return _combined(clip_remap, clip_emb, num_list)


# speedup vs baseline: 1.1461x; 1.0112x over previous
import jax
import jax.numpy as jnp
from jax import lax
from jax.experimental import pallas as pl
from jax.experimental.pallas import tpu as pltpu
from jax.experimental.pallas import tpu_sc as plsc

NC = 2
NS = 16
NW = NC * NS
L = 16
RB = 16
D = 512
CHUNKS = D // L

TC_ROWS = 9216
BR = 512


def _sc_body(remap_hbm, emb_hbm, nl_hbm, out_hbm,
             nl_ref, acc_ref, mat_ref,
             br0, be0, br1, be1, br2, be2, br3, be3,
             sem0, sem1, sem2, sem3):
    cid = lax.axis_index("c")
    sid = lax.axis_index("s")
    wid = cid * NS + sid

    pltpu.sync_copy(nl_hbm, nl_ref)
    nl = nl_ref[...]
    wseg_vec = 1.0 / jnp.maximum(nl, 1).astype(jnp.float32)
    cum = []
    run = jnp.int32(0)
    wseg = []
    for j in range(16):
        run = run + nl[j]
        cum.append(run)
        wseg.append(wseg_vec[j])
    total = cum[15]
    rem = jnp.maximum(total - TC_ROWS, 0)
    nb = (rem + RB - 1) >> 4
    nmy = (jnp.maximum(nb - wid, 0) + (NW - 1)) // NW

    def copies(i, br, be, sem):
        row0 = TC_ROWS + (wid + i * NW) * RB
        cr = pltpu.make_async_copy(remap_hbm.at[pl.ds(row0, RB)], br, sem)
        ce = pltpu.make_async_copy(emb_hbm.at[pl.ds(row0, RB)], be, sem)
        return cr, ce

    def issue(i, br, be, sem):
        cr, ce = copies(i, br, be, sem)
        cr.start()
        ce.start()

    def drain(i, br, be, sem):
        cr, ce = copies(i, br, be, sem)
        cr.wait()
        ce.wait()

    lane = lax.iota(jnp.int32, L)

    def compute_block(br, be, i):
        row0 = TC_ROWS + (wid + i * NW) * RB

        def row_body(r, carry):
            a16 = jnp.zeros((L,), jnp.float32)
            for c in range(CHUNKS):
                a = br[r, pl.ds(c * L, L)]
                b = be[r, pl.ds(c * L, L)]
                d = a - b
                a16 = a16 + d * d
            mat_ref[r, :] = a16
            return carry
        lax.fori_loop(0, RB, row_body, 0, unroll=2)

        ssq = jnp.zeros((L,), jnp.float32)
        for c in range(L):
            col = jnp.zeros((L,), jnp.int32) + c
            ssq = ssq + plsc.load_gather(mat_ref, [lane, col])

        s = jnp.maximum(ssq, 1e-30)
        ii = plsc.bitcast(s, jnp.int32)
        y = plsc.bitcast(jnp.int32(0x5F3759DF) - (ii >> 1), jnp.float32)
        for _ in range(3):
            y = y * (1.5 - 0.5 * s * y * y)
        norm = s * y

        p = row0 + lane
        w = jnp.zeros((L,), jnp.float32) + wseg[0]
        for j in range(1, 16):
            w = w + jnp.where(p >= cum[j - 1], wseg[j] - wseg[j - 1], 0.0)
        w = jnp.where(p >= total, 0.0, w)
        return w * norm

    slots = ((br0, be0, sem0), (br1, be1, sem1),
             (br2, be2, sem2), (br3, be3, sem3))

    for s in range(3):
        @pl.when(s < nmy)
        def _(s=s):
            issue(s, *slots[s])

    def sub_block(i, u, acc):
        br, be, sem = slots[u]

        def go(a):
            drain(i, br, be, sem)

            @pl.when(i + 3 < nmy)
            def _():
                issue(i + 3, *slots[(u + 3) % 4])

            return a + compute_block(br, be, i)

        return lax.cond(i < nmy, go, lambda a: a, acc)

    def quad_body(q, acc):
        i0 = 4 * q
        for u in range(4):
            acc = sub_block(i0 + u, u, acc)
        return acc

    nquads = (nmy + 3) >> 2
    acc = lax.fori_loop(0, nquads, quad_body, jnp.zeros((L,), jnp.float32))

    acc_ref[...] = acc
    pltpu.sync_copy(acc_ref, out_hbm.at[wid])


def _tc_body(nl_ref, a_ref, b_ref, o_ref):
    i = pl.program_id(0)
    d = a_ref[...] - b_ref[...]
    s = jnp.sum(d * d, axis=1).reshape(BR // 128, 128)
    norm = jnp.sqrt(s)

    p = (i * BR
         + lax.broadcasted_iota(jnp.int32, (BR // 128, 128), 0) * 128
         + lax.broadcasted_iota(jnp.int32, (BR // 128, 128), 1))

    w = jnp.full((BR // 128, 128), 1.0 / jnp.maximum(nl_ref[0], 1).astype(jnp.float32))
    off = nl_ref[0]
    for j in range(1, 16):
        wj = 1.0 / jnp.maximum(nl_ref[j], 1).astype(jnp.float32)
        w = jnp.where(p >= off, wj, w)
        off = off + nl_ref[j]
    w = jnp.where(p >= off, 0.0, w)

    @pl.when(i == 0)
    def _():
        o_ref[...] = jnp.zeros_like(o_ref)

    o_ref[...] += jnp.sum(w * norm).reshape(1, 1)


@jax.jit
def _combined(clip_remap, clip_emb, num_list):
    mesh = plsc.VectorSubcoreMesh(core_axis_name="c", subcore_axis_name="s",
                                  num_cores=NC, num_subcores=NS)
    sc = pl.kernel(
        _sc_body,
        out_type=jax.ShapeDtypeStruct((NW, L), jnp.float32),
        mesh=mesh,
        compiler_params=pltpu.CompilerParams(needs_layout_passes=False),
        scratch_types=[
            pltpu.VMEM((16,), jnp.int32),
            pltpu.VMEM((L,), jnp.float32),
            pltpu.VMEM((RB, L), jnp.float32),
            pltpu.VMEM((RB, D), jnp.float32),
            pltpu.VMEM((RB, D), jnp.float32),
            pltpu.VMEM((RB, D), jnp.float32),
            pltpu.VMEM((RB, D), jnp.float32),
            pltpu.VMEM((RB, D), jnp.float32),
            pltpu.VMEM((RB, D), jnp.float32),
            pltpu.VMEM((RB, D), jnp.float32),
            pltpu.VMEM((RB, D), jnp.float32),
            pltpu.SemaphoreType.DMA,
            pltpu.SemaphoreType.DMA,
            pltpu.SemaphoreType.DMA,
            pltpu.SemaphoreType.DMA,
        ],
    )(clip_remap, clip_emb, num_list)

    tc = pl.pallas_call(
        _tc_body,
        grid=(TC_ROWS // BR,),
        in_specs=[
            pl.BlockSpec(memory_space=pltpu.SMEM),
            pl.BlockSpec((BR, D), lambda i: (i, 0)),
            pl.BlockSpec((BR, D), lambda i: (i, 0)),
        ],
        out_specs=pl.BlockSpec((1, 1), lambda i: (0, 0)),
        out_shape=jax.ShapeDtypeStruct((1, 1), jnp.float32),
        compiler_params=pltpu.CompilerParams(
            dimension_semantics=("arbitrary",)),
    )(num_list, clip_remap, clip_emb)

    return tc[0, 0] + jnp.sum(sc)


def kernel(clip_remap, clip_emb, num_list):
    return _combined(clip_remap, clip_emb, num_list)
